# baseline (speedup 1.0000x reference)
import functools

import jax
import jax.numpy as jnp
from jax import lax
from jax.experimental import pallas as pl
from jax.experimental.pallas import tpu as pltpu
from jax.experimental.pallas import tpu_sc as plsc

N_NODES = 10000
N_EDGES = 320000
D = 128

NC = 2
NS = 16
NW = NC * NS

EPT = N_EDGES // NW
K = 128
NCHUNK = 80
EPT_PAD = NCHUNK * K
NB = 2
IQG = 8

H_BINS = 10240
H_R = 80
A_ROWS_T = 632
A_ROWS = A_ROWS_T * NS

_mesh = plsc.VectorSubcoreMesh(core_axis_name="c", subcore_axis_name="s")


def _wid():
  return lax.axis_index("s") * NC + lax.axis_index("c")


@functools.partial(
    pl.kernel,
    out_type=jax.ShapeDtypeStruct((NC, H_R, 128), jnp.float32),
    mesh=_mesh,
    scratch_types=[
        pltpu.VMEM((EPT,), jnp.int32),
        pltpu.VMEM((1, H_R), jnp.int32),
        pltpu.VMEM((H_R, 128), jnp.float32),
        pltpu.VMEM_SHARED((H_R, 128), jnp.float32),
    ],
    compiler_params=pltpu.CompilerParams(needs_layout_passes=False),
)
def _deg_kernel(dst2_hbm, iota_hbm, z_hbm, deg_out, idx_v, iota_v, hist_v,
                hist_sh):
  cid = lax.axis_index("c")
  sid = lax.axis_index("s")
  wid = _wid()
  pltpu.sync_copy(iota_hbm, iota_v)
  pltpu.sync_copy(z_hbm, hist_v)
  pltpu.sync_copy(dst2_hbm.at[wid], idx_v)

  @pl.when(sid == 0)
  def _():
    pltpu.sync_copy(z_hbm, hist_sh)

  ones16 = jnp.ones((16,), jnp.float32)

  def grp(j, carry):
    idx = idx_v[pl.ds(j * 16, 16)]
    row = lax.shift_right_logical(idx, 7)
    col = lax.bitwise_and(idx, 127)
    plsc.addupdate_scatter(hist_v, [row, col], ones16)
    return carry

  lax.fori_loop(0, EPT // 16, grp, 0)
  plsc.subcore_barrier()
  pltpu.sync_copy(hist_v, hist_sh.at[iota_v.at[0]], add=True)
  plsc.subcore_barrier()

  @pl.when(sid == 0)
  def _():
    pltpu.sync_copy(hist_sh, deg_out.at[cid])


def _scale_body(degp_ref, x_ref, v_ref):
  deg = degp_ref[0, :N_NODES] + degp_ref[1, :N_NODES] + 1.0
  dinv = lax.rsqrt(deg)
  v_ref[...] = x_ref[...] * dinv[:, None]


def _scale_call(degp, x):
  return pl.pallas_call(
      _scale_body,
      out_shape=jax.ShapeDtypeStruct((N_NODES, D), jnp.float32),
  )(degp, x)


@functools.partial(
    pl.kernel,
    out_type=jax.ShapeDtypeStruct((NC, A_ROWS, D), jnp.float32),
    mesh=_mesh,
    scratch_types=[
        pltpu.VMEM((NCHUNK, K), jnp.int32),
        pltpu.VMEM((NCHUNK, K), jnp.int32),
        pltpu.VMEM((K, D), jnp.float32),
        pltpu.VMEM_SHARED((A_ROWS, D), jnp.float32),
    ],
)
def _edge_kernel(src3_hbm, dst3_hbm, v_hbm, z_hbm, acc_out,
                 idxd_v, idxs_v, rows_v, acc_sh):
  cid = lax.axis_index("c")
  sid = lax.axis_index("s")
  wid = _wid()
  pltpu.sync_copy(z_hbm, acc_sh.at[pl.ds(sid * A_ROWS_T, A_ROWS_T)])
  pltpu.sync_copy(dst3_hbm.at[wid], idxd_v)
  pltpu.sync_copy(src3_hbm.at[wid], idxs_v)
  plsc.subcore_barrier()

  def body(c, carry):
    pltpu.sync_copy(v_hbm.at[idxs_v.at[c]], rows_v)
    pltpu.sync_copy(rows_v, acc_sh.at[idxd_v.at[c]], add=True)
    return carry

  lax.fori_loop(0, NCHUNK, body, 0)
  plsc.subcore_barrier()
  pltpu.sync_copy(
      acc_sh.at[pl.ds(sid * A_ROWS_T, A_ROWS_T)],
      acc_out.at[cid, pl.ds(sid * A_ROWS_T, A_ROWS_T)],
  )


def _out_body(accp_ref, degp_ref, x_ref, w_ref, b_ref, o_ref):
  deg = degp_ref[0, :N_NODES] + degp_ref[1, :N_NODES] + 1.0
  dinv = lax.rsqrt(deg)
  s = accp_ref[0, :N_NODES] + accp_ref[1, :N_NODES] + x_ref[...] * dinv[:, None]
  t = s * dinv[:, None]
  o_ref[...] = (
      jnp.dot(t, w_ref[...], preferred_element_type=jnp.float32)
      + b_ref[...][None, :]
  )


def _out_call(accp, degp, x, W, b):
  return pl.pallas_call(
      _out_body,
      out_shape=jax.ShapeDtypeStruct((N_NODES, D), jnp.float32),
  )(accp, degp, x, W, b)


def kernel(x, edge_index, W, b):
  ei = edge_index.astype(jnp.int32)
  src = ei[0]
  dst = ei[1]
  pad_n = NW * EPT_PAD - N_EDGES
  src3 = jnp.concatenate(
      [src, jnp.zeros((pad_n,), jnp.int32)]).reshape(NW, NCHUNK, K)
  dst3 = jnp.concatenate(
      [dst, jnp.full((pad_n,), A_ROWS - 2, jnp.int32)]).reshape(NW, NCHUNK, K)
  dst2 = dst.reshape(NW, EPT)
  iota_r = jnp.arange(H_R, dtype=jnp.int32).reshape(1, H_R)
  z_hist = jnp.zeros((H_R, 128), jnp.float32)
  z_acc = jnp.zeros((A_ROWS_T, D), jnp.float32)

  degp = _deg_kernel(dst2, iota_r, z_hist)
  degf = degp.reshape(NC, H_BINS)
  v = _scale_call(degf, x)
  accp = _edge_kernel(src3, dst3, v, z_acc)
  return _out_call(accp, degf, x, W, b)


# speedup vs baseline: 2.3053x; 2.3053x over previous
import functools

import jax
import jax.numpy as jnp
from jax import lax
from jax.experimental import pallas as pl
from jax.experimental.pallas import tpu as pltpu
from jax.experimental.pallas import tpu_sc as plsc

N_NODES = 10000
N_EDGES = 320000
D = 128

NC = 2
NS = 16
NW = NC * NS

EPT = N_EDGES // NW
K = 80
NCHUNK = 125

H_BINS = 10240
H_R = 80
A_ROWS_T = 632
A_ROWS = A_ROWS_T * NS

_mesh = plsc.VectorSubcoreMesh(core_axis_name="c", subcore_axis_name="s")


def _wid():
  return lax.axis_index("s") * NC + lax.axis_index("c")


@functools.partial(
    pl.kernel,
    out_type=jax.ShapeDtypeStruct((NC, H_R, 128), jnp.float32),
    mesh=_mesh,
    scratch_types=[
        pltpu.VMEM((EPT,), jnp.int32),
        pltpu.VMEM((1, H_R), jnp.int32),
        pltpu.VMEM((H_R, 128), jnp.float32),
        pltpu.VMEM_SHARED((H_R, 128), jnp.float32),
    ],
    compiler_params=pltpu.CompilerParams(needs_layout_passes=False),
)
def _deg_kernel(dst2_hbm, iota_hbm, z_hbm, deg_out, idx_v, iota_v, hist_v,
                hist_sh):
  cid = lax.axis_index("c")
  sid = lax.axis_index("s")
  wid = _wid()
  pltpu.sync_copy(iota_hbm, iota_v)
  pltpu.sync_copy(z_hbm, hist_v)
  pltpu.sync_copy(dst2_hbm.at[wid], idx_v)

  @pl.when(sid == 0)
  def _():
    pltpu.sync_copy(z_hbm, hist_sh)

  ones16 = jnp.ones((16,), jnp.float32)

  def grp(j, carry):
    idx = idx_v[pl.ds(j * 16, 16)]
    row = lax.shift_right_logical(idx, 7)
    col = lax.bitwise_and(idx, 127)
    plsc.addupdate_scatter(hist_v, [row, col], ones16)
    return carry

  lax.fori_loop(0, EPT // 16, grp, 0)
  plsc.subcore_barrier()
  pltpu.sync_copy(hist_v, hist_sh.at[iota_v.at[0]], add=True)
  plsc.subcore_barrier()

  @pl.when(sid == 0)
  def _():
    pltpu.sync_copy(hist_sh, deg_out.at[cid])


def _scale_body(degp_ref, x_ref, v_ref):
  deg = degp_ref[0, :N_NODES] + degp_ref[1, :N_NODES] + 1.0
  dinv = lax.rsqrt(deg)
  v_ref[...] = x_ref[...] * dinv[:, None]


def _scale_call(degp, x):
  return pl.pallas_call(
      _scale_body,
      out_shape=jax.ShapeDtypeStruct((N_NODES, D), jnp.float32),
  )(degp, x)


@functools.partial(
    pl.kernel,
    out_type=jax.ShapeDtypeStruct((NC, A_ROWS, D), jnp.float32),
    mesh=_mesh,
    scratch_types=[
        pltpu.VMEM((NCHUNK, K), jnp.int32),
        pltpu.VMEM((NCHUNK, K), jnp.int32),
        pltpu.VMEM((K, D), jnp.float32),
        pltpu.VMEM_SHARED((A_ROWS, D), jnp.float32),
    ],
)
def _edge_kernel(src3_hbm, dst3_hbm, v_hbm, z_hbm, acc_out,
                 idxd_v, idxs_v, rows_v, acc_sh):
  cid = lax.axis_index("c")
  sid = lax.axis_index("s")
  wid = _wid()
  pltpu.sync_copy(z_hbm, acc_sh.at[pl.ds(sid * A_ROWS_T, A_ROWS_T)])
  pltpu.sync_copy(dst3_hbm.at[wid], idxd_v)
  pltpu.sync_copy(src3_hbm.at[wid], idxs_v)
  plsc.subcore_barrier()

  def body(c, carry):
    pltpu.sync_copy(v_hbm.at[idxs_v.at[c]], rows_v)
    pltpu.sync_copy(rows_v, acc_sh.at[idxd_v.at[c]], add=True)
    return carry

  lax.fori_loop(0, NCHUNK, body, 0)
  plsc.subcore_barrier()
  pltpu.sync_copy(
      acc_sh.at[pl.ds(sid * A_ROWS_T, A_ROWS_T)],
      acc_out.at[cid, pl.ds(sid * A_ROWS_T, A_ROWS_T)],
  )


def _out_body(accp_ref, degp_ref, x_ref, w_ref, b_ref, o_ref):
  deg = degp_ref[0, :N_NODES] + degp_ref[1, :N_NODES] + 1.0
  dinv = lax.rsqrt(deg)
  s = accp_ref[0, :N_NODES] + accp_ref[1, :N_NODES] + x_ref[...] * dinv[:, None]
  t = s * dinv[:, None]
  o_ref[...] = (
      jnp.dot(t, w_ref[...], preferred_element_type=jnp.float32)
      + b_ref[...][None, :]
  )


def _out_call(accp, degp, x, W, b):
  return pl.pallas_call(
      _out_body,
      out_shape=jax.ShapeDtypeStruct((N_NODES, D), jnp.float32),
  )(accp, degp, x, W, b)


def kernel(x, edge_index, W, b):
  ei = edge_index.astype(jnp.int32)
  src = ei[0]
  dst = ei[1]
  src3 = src.reshape(NW, NCHUNK, K)
  dst3 = dst.reshape(NW, NCHUNK, K)
  dst2 = dst.reshape(NW, EPT)
  iota_r = jnp.arange(H_R, dtype=jnp.int32).reshape(1, H_R)
  z_hist = jnp.zeros((H_R, 128), jnp.float32)
  z_acc = jnp.zeros((A_ROWS_T, D), jnp.float32)

  degp = _deg_kernel(dst2, iota_r, z_hist)
  degf = degp.reshape(NC, H_BINS)
  v = _scale_call(degf, x)
  accp = _edge_kernel(src3, dst3, v, z_acc)
  return _out_call(accp, degf, x, W, b)


# speedup vs baseline: 2.4609x; 1.0675x over previous
import functools

import jax
import jax.numpy as jnp
from jax import lax
from jax.experimental import pallas as pl
from jax.experimental.pallas import tpu as pltpu
from jax.experimental.pallas import tpu_sc as plsc

N_NODES = 10000
N_EDGES = 320000
D = 128

NC = 2
NS = 16
NW = NC * NS

EPT = N_EDGES // NW
K = 100
NCHUNK = 100

H_BINS = 10240
H_R = 80
A_ROWS_T = 632
A_ROWS = A_ROWS_T * NS

_mesh = plsc.VectorSubcoreMesh(core_axis_name="c", subcore_axis_name="s")


def _wid():
  return lax.axis_index("s") * NC + lax.axis_index("c")


@functools.partial(
    pl.kernel,
    out_type=jax.ShapeDtypeStruct((NC, H_R, 128), jnp.float32),
    mesh=_mesh,
    scratch_types=[
        pltpu.VMEM((EPT,), jnp.int32),
        pltpu.VMEM((1, H_R), jnp.int32),
        pltpu.VMEM((H_R, 128), jnp.float32),
        pltpu.VMEM_SHARED((H_R, 128), jnp.float32),
    ],
    compiler_params=pltpu.CompilerParams(needs_layout_passes=False),
)
def _deg_kernel(dst2_hbm, iota_hbm, z_hbm, deg_out, idx_v, iota_v, hist_v,
                hist_sh):
  cid = lax.axis_index("c")
  sid = lax.axis_index("s")
  wid = _wid()
  pltpu.sync_copy(iota_hbm, iota_v)
  pltpu.sync_copy(z_hbm, hist_v)
  pltpu.sync_copy(dst2_hbm.at[wid], idx_v)

  @pl.when(sid == 0)
  def _():
    pltpu.sync_copy(z_hbm, hist_sh)

  ones16 = jnp.ones((16,), jnp.float32)

  def grp(j, carry):
    idx = idx_v[pl.ds(j * 16, 16)]
    row = lax.shift_right_logical(idx, 7)
    col = lax.bitwise_and(idx, 127)
    plsc.addupdate_scatter(hist_v, [row, col], ones16)
    return carry

  lax.fori_loop(0, EPT // 16, grp, 0)
  plsc.subcore_barrier()
  pltpu.sync_copy(hist_v, hist_sh.at[iota_v.at[0]], add=True)
  plsc.subcore_barrier()

  @pl.when(sid == 0)
  def _():
    pltpu.sync_copy(hist_sh, deg_out.at[cid])


def _scale_body(degp_ref, x_ref, v_ref):
  deg = degp_ref[0, :N_NODES] + degp_ref[1, :N_NODES] + 1.0
  dinv = lax.rsqrt(deg)
  v_ref[...] = x_ref[...] * dinv[:, None]


def _scale_call(degp, x):
  return pl.pallas_call(
      _scale_body,
      out_shape=jax.ShapeDtypeStruct((N_NODES, D), jnp.float32),
  )(degp, x)


@functools.partial(
    pl.kernel,
    out_type=jax.ShapeDtypeStruct((NC, A_ROWS, D), jnp.float32),
    mesh=_mesh,
    scratch_types=[
        pltpu.VMEM((NCHUNK, K), jnp.int32),
        pltpu.VMEM((NCHUNK, K), jnp.int32),
        pltpu.VMEM((K, D), jnp.float32),
        pltpu.VMEM_SHARED((A_ROWS, D), jnp.float32),
    ],
)
def _edge_kernel(src3_hbm, dst3_hbm, v_hbm, z_hbm, acc_out,
                 idxd_v, idxs_v, rows_v, acc_sh):
  cid = lax.axis_index("c")
  sid = lax.axis_index("s")
  wid = _wid()
  pltpu.sync_copy(z_hbm, acc_sh.at[pl.ds(sid * A_ROWS_T, A_ROWS_T)])
  pltpu.sync_copy(dst3_hbm.at[wid], idxd_v)
  pltpu.sync_copy(src3_hbm.at[wid], idxs_v)
  plsc.subcore_barrier()

  def body(c, carry):
    pltpu.sync_copy(v_hbm.at[idxs_v.at[c]], rows_v)
    pltpu.sync_copy(rows_v, acc_sh.at[idxd_v.at[c]], add=True)
    return carry

  lax.fori_loop(0, NCHUNK, body, 0)
  plsc.subcore_barrier()
  pltpu.sync_copy(
      acc_sh.at[pl.ds(sid * A_ROWS_T, A_ROWS_T)],
      acc_out.at[cid, pl.ds(sid * A_ROWS_T, A_ROWS_T)],
  )


def _out_body(accp_ref, degp_ref, x_ref, w_ref, b_ref, o_ref):
  deg = degp_ref[0, :N_NODES] + degp_ref[1, :N_NODES] + 1.0
  dinv = lax.rsqrt(deg)
  s = accp_ref[0, :N_NODES] + accp_ref[1, :N_NODES] + x_ref[...] * dinv[:, None]
  t = s * dinv[:, None]
  o_ref[...] = (
      jnp.dot(t, w_ref[...], preferred_element_type=jnp.float32)
      + b_ref[...][None, :]
  )


def _out_call(accp, degp, x, W, b):
  return pl.pallas_call(
      _out_body,
      out_shape=jax.ShapeDtypeStruct((N_NODES, D), jnp.float32),
  )(accp, degp, x, W, b)


def kernel(x, edge_index, W, b):
  ei = edge_index.astype(jnp.int32)
  src = ei[0]
  dst = ei[1]
  src3 = src.reshape(NW, NCHUNK, K)
  dst3 = dst.reshape(NW, NCHUNK, K)
  dst2 = dst.reshape(NW, EPT)
  iota_r = jnp.arange(H_R, dtype=jnp.int32).reshape(1, H_R)
  z_hist = jnp.zeros((H_R, 128), jnp.float32)
  z_acc = jnp.zeros((A_ROWS_T, D), jnp.float32)

  degp = _deg_kernel(dst2, iota_r, z_hist)
  degf = degp.reshape(NC, H_BINS)
  v = _scale_call(degf, x)
  accp = _edge_kernel(src3, dst3, v, z_acc)
  return _out_call(accp, degf, x, W, b)


# speedup vs baseline: 2.6089x; 1.0602x over previous
import functools

import jax
import jax.numpy as jnp
from jax import lax
from jax.experimental import pallas as pl
from jax.experimental.pallas import tpu as pltpu
from jax.experimental.pallas import tpu_sc as plsc

N_NODES = 10000
N_EDGES = 320000
D = 128

NC = 2
NS = 16
NW = NC * NS

EPT = N_EDGES // NW
K = 125
NCHUNK = 80

H_BINS = 10240
H_R = 80
A_ROWS_T = 632
A_ROWS = A_ROWS_T * NS

_mesh = plsc.VectorSubcoreMesh(core_axis_name="c", subcore_axis_name="s")


def _wid():
  return lax.axis_index("s") * NC + lax.axis_index("c")


@functools.partial(
    pl.kernel,
    out_type=jax.ShapeDtypeStruct((NC, H_R, 128), jnp.float32),
    mesh=_mesh,
    scratch_types=[
        pltpu.VMEM((EPT,), jnp.int32),
        pltpu.VMEM((1, H_R), jnp.int32),
        pltpu.VMEM((H_R, 128), jnp.float32),
        pltpu.VMEM_SHARED((H_R, 128), jnp.float32),
    ],
    compiler_params=pltpu.CompilerParams(needs_layout_passes=False),
)
def _deg_kernel(dst2_hbm, iota_hbm, z_hbm, deg_out, idx_v, iota_v, hist_v,
                hist_sh):
  cid = lax.axis_index("c")
  sid = lax.axis_index("s")
  wid = _wid()
  pltpu.sync_copy(iota_hbm, iota_v)
  pltpu.sync_copy(z_hbm, hist_v)
  pltpu.sync_copy(dst2_hbm.at[wid], idx_v)

  @pl.when(sid == 0)
  def _():
    pltpu.sync_copy(z_hbm, hist_sh)

  ones16 = jnp.ones((16,), jnp.float32)

  def grp(j, carry):
    idx = idx_v[pl.ds(j * 16, 16)]
    row = lax.shift_right_logical(idx, 7)
    col = lax.bitwise_and(idx, 127)
    plsc.addupdate_scatter(hist_v, [row, col], ones16)
    return carry

  lax.fori_loop(0, EPT // 16, grp, 0)
  plsc.subcore_barrier()
  pltpu.sync_copy(hist_v, hist_sh.at[iota_v.at[0]], add=True)
  plsc.subcore_barrier()

  @pl.when(sid == 0)
  def _():
    pltpu.sync_copy(hist_sh, deg_out.at[cid])


def _scale_body(degp_ref, x_ref, v_ref):
  deg = degp_ref[0, :N_NODES] + degp_ref[1, :N_NODES] + 1.0
  dinv = lax.rsqrt(deg)
  v_ref[...] = x_ref[...] * dinv[:, None]


def _scale_call(degp, x):
  return pl.pallas_call(
      _scale_body,
      out_shape=jax.ShapeDtypeStruct((N_NODES, D), jnp.float32),
  )(degp, x)


@functools.partial(
    pl.kernel,
    out_type=jax.ShapeDtypeStruct((NC, A_ROWS, D), jnp.float32),
    mesh=_mesh,
    scratch_types=[
        pltpu.VMEM((NCHUNK, K), jnp.int32),
        pltpu.VMEM((NCHUNK, K), jnp.int32),
        pltpu.VMEM((K, D), jnp.float32),
        pltpu.VMEM_SHARED((A_ROWS, D), jnp.float32),
    ],
)
def _edge_kernel(src3_hbm, dst3_hbm, v_hbm, z_hbm, acc_out,
                 idxd_v, idxs_v, rows_v, acc_sh):
  cid = lax.axis_index("c")
  sid = lax.axis_index("s")
  wid = _wid()
  pltpu.sync_copy(z_hbm, acc_sh.at[pl.ds(sid * A_ROWS_T, A_ROWS_T)])
  pltpu.sync_copy(dst3_hbm.at[wid], idxd_v)
  pltpu.sync_copy(src3_hbm.at[wid], idxs_v)
  plsc.subcore_barrier()

  def body(c, carry):
    pltpu.sync_copy(v_hbm.at[idxs_v.at[c]], rows_v)
    pltpu.sync_copy(rows_v, acc_sh.at[idxd_v.at[c]], add=True)
    return carry

  lax.fori_loop(0, NCHUNK, body, 0)
  plsc.subcore_barrier()
  pltpu.sync_copy(
      acc_sh.at[pl.ds(sid * A_ROWS_T, A_ROWS_T)],
      acc_out.at[cid, pl.ds(sid * A_ROWS_T, A_ROWS_T)],
  )


def _out_body(accp_ref, degp_ref, x_ref, w_ref, b_ref, o_ref):
  deg = degp_ref[0, :N_NODES] + degp_ref[1, :N_NODES] + 1.0
  dinv = lax.rsqrt(deg)
  s = accp_ref[0, :N_NODES] + accp_ref[1, :N_NODES] + x_ref[...] * dinv[:, None]
  t = s * dinv[:, None]
  o_ref[...] = (
      jnp.dot(t, w_ref[...], preferred_element_type=jnp.float32)
      + b_ref[...][None, :]
  )


def _out_call(accp, degp, x, W, b):
  return pl.pallas_call(
      _out_body,
      out_shape=jax.ShapeDtypeStruct((N_NODES, D), jnp.float32),
  )(accp, degp, x, W, b)


def kernel(x, edge_index, W, b):
  ei = edge_index.astype(jnp.int32)
  src = ei[0]
  dst = ei[1]
  src3 = src.reshape(NW, NCHUNK, K)
  dst3 = dst.reshape(NW, NCHUNK, K)
  dst2 = dst.reshape(NW, EPT)
  iota_r = jnp.arange(H_R, dtype=jnp.int32).reshape(1, H_R)
  z_hist = jnp.zeros((H_R, 128), jnp.float32)
  z_acc = jnp.zeros((A_ROWS_T, D), jnp.float32)

  degp = _deg_kernel(dst2, iota_r, z_hist)
  degf = degp.reshape(NC, H_BINS)
  v = _scale_call(degf, x)
  accp = _edge_kernel(src3, dst3, v, z_acc)
  return _out_call(accp, degf, x, W, b)


# speedup vs baseline: 3.6077x; 1.3828x over previous
import functools

import jax
import jax.numpy as jnp
from jax import lax
from jax.experimental import pallas as pl
from jax.experimental.pallas import tpu as pltpu
from jax.experimental.pallas import tpu_sc as plsc

N_NODES = 10000
N_EDGES = 320000
D = 128

NC = 2
NS = 16
NW = NC * NS

EPT = N_EDGES // NW
K = 125
NCHUNK = 80

H_BINS = 10240
H_R = 80
A_ROWS_T = 632
A_ROWS = A_ROWS_T * NS

_mesh = plsc.VectorSubcoreMesh(core_axis_name="c", subcore_axis_name="s")


def _wid():
  return lax.axis_index("s") * NC + lax.axis_index("c")


@functools.partial(
    pl.kernel,
    out_type=jax.ShapeDtypeStruct((NC, H_R, 128), jnp.float32),
    mesh=_mesh,
    scratch_types=[
        pltpu.VMEM((EPT,), jnp.int32),
        pltpu.VMEM((1, H_R), jnp.int32),
        pltpu.VMEM((H_R, 128), jnp.float32),
        pltpu.VMEM_SHARED((H_R, 128), jnp.float32),
    ],
    compiler_params=pltpu.CompilerParams(needs_layout_passes=False),
)
def _deg_kernel(dst2_hbm, iota_hbm, z_hbm, deg_out, idx_v, iota_v, hist_v,
                hist_sh):
  cid = lax.axis_index("c")
  sid = lax.axis_index("s")
  wid = _wid()
  pltpu.sync_copy(iota_hbm, iota_v)
  pltpu.sync_copy(z_hbm, hist_v)
  pltpu.sync_copy(dst2_hbm.at[wid], idx_v)

  @pl.when(sid == 0)
  def _():
    pltpu.sync_copy(z_hbm, hist_sh)

  ones16 = jnp.ones((16,), jnp.float32)

  def grp(j, carry):
    idx = idx_v[pl.ds(j * 16, 16)]
    row = lax.shift_right_logical(idx, 7)
    col = lax.bitwise_and(idx, 127)
    plsc.addupdate_scatter(hist_v, [row, col], ones16)
    return carry

  lax.fori_loop(0, EPT // 16, grp, 0)
  plsc.subcore_barrier()
  pltpu.sync_copy(hist_v, hist_sh.at[iota_v.at[0]], add=True)
  plsc.subcore_barrier()

  @pl.when(sid == 0)
  def _():
    pltpu.sync_copy(hist_sh, deg_out.at[cid])


def _scale_body(degp_ref, x_ref, v_ref):
  deg = degp_ref[0, :N_NODES] + degp_ref[1, :N_NODES] + 1.0
  dinv = lax.rsqrt(deg)
  v_ref[...] = x_ref[...] * dinv[:, None]


def _scale_call(degp, x):
  return pl.pallas_call(
      _scale_body,
      out_shape=jax.ShapeDtypeStruct((N_NODES, D), jnp.float32),
  )(degp, x)


@functools.partial(
    pl.kernel,
    out_type=jax.ShapeDtypeStruct((NC, A_ROWS, D), jnp.float32),
    mesh=_mesh,
    scratch_types=[
        pltpu.VMEM((NCHUNK // 2, K), jnp.int32),
        pltpu.VMEM((NCHUNK // 2, K), jnp.int32),
        pltpu.VMEM((2, K, D), jnp.float32),
        pltpu.SemaphoreType.DMA,
        pltpu.SemaphoreType.DMA,
        pltpu.VMEM_SHARED((A_ROWS, D), jnp.float32),
    ],
)
def _edge_kernel(src3_hbm, dst3_hbm, v_hbm, z_hbm, acc_out,
                 idxd_v, idxs_v, rows_v, gsem0, gsem1, acc_sh):
  cid = lax.axis_index("c")
  sid = lax.axis_index("s")
  wid = _wid()
  gsem = [gsem0, gsem1]
  HALF = NCHUNK // 2
  pltpu.sync_copy(z_hbm, acc_sh.at[pl.ds(sid * A_ROWS_T, A_ROWS_T)])
  plsc.subcore_barrier()

  for h in range(2):
    pltpu.sync_copy(dst3_hbm.at[wid, pl.ds(h * HALF, HALF)], idxd_v)
    pltpu.sync_copy(src3_hbm.at[wid, pl.ds(h * HALF, HALF)], idxs_v)
    pltpu.async_copy(v_hbm.at[idxs_v.at[0]], rows_v.at[0], gsem0)
    pltpu.async_copy(v_hbm.at[idxs_v.at[1]], rows_v.at[1], gsem1)

    def body(cc, carry):
      for rb in range(2):
        c = cc * 2 + rb
        pltpu.make_async_copy(
            v_hbm.at[idxs_v.at[c]], rows_v.at[rb], gsem[rb]).wait()
        pltpu.sync_copy(rows_v.at[rb], acc_sh.at[idxd_v.at[c]], add=True)

        @pl.when(c + 2 < HALF)
        def _():
          pltpu.async_copy(
              v_hbm.at[idxs_v.at[c + 2]], rows_v.at[rb], gsem[rb])
      return carry

    lax.fori_loop(0, HALF // 2, body, 0)

  plsc.subcore_barrier()
  pltpu.sync_copy(
      acc_sh.at[pl.ds(sid * A_ROWS_T, A_ROWS_T)],
      acc_out.at[cid, pl.ds(sid * A_ROWS_T, A_ROWS_T)],
  )


def _out_body(accp_ref, degp_ref, x_ref, w_ref, b_ref, o_ref):
  deg = degp_ref[0, :N_NODES] + degp_ref[1, :N_NODES] + 1.0
  dinv = lax.rsqrt(deg)
  s = accp_ref[0, :N_NODES] + accp_ref[1, :N_NODES] + x_ref[...] * dinv[:, None]
  t = s * dinv[:, None]
  o_ref[...] = (
      jnp.dot(t, w_ref[...], preferred_element_type=jnp.float32)
      + b_ref[...][None, :]
  )


def _out_call(accp, degp, x, W, b):
  return pl.pallas_call(
      _out_body,
      out_shape=jax.ShapeDtypeStruct((N_NODES, D), jnp.float32),
  )(accp, degp, x, W, b)


def kernel(x, edge_index, W, b):
  ei = edge_index.astype(jnp.int32)
  src = ei[0]
  dst = ei[1]
  src3 = src.reshape(NW, NCHUNK, K)
  dst3 = dst.reshape(NW, NCHUNK, K)
  dst2 = dst.reshape(NW, EPT)
  iota_r = jnp.arange(H_R, dtype=jnp.int32).reshape(1, H_R)
  z_hist = jnp.zeros((H_R, 128), jnp.float32)
  z_acc = jnp.zeros((A_ROWS_T, D), jnp.float32)

  degp = _deg_kernel(dst2, iota_r, z_hist)
  degf = degp.reshape(NC, H_BINS)
  v = _scale_call(degf, x)
  accp = _edge_kernel(src3, dst3, v, z_acc)
  return _out_call(accp, degf, x, W, b)


# speedup vs baseline: 3.6102x; 1.0007x over previous
import functools

import jax
import jax.numpy as jnp
from jax import lax
from jax.experimental import pallas as pl
from jax.experimental.pallas import tpu as pltpu
from jax.experimental.pallas import tpu_sc as plsc

N_NODES = 10000
N_EDGES = 320000
D = 128

NC = 2
NS = 16
NW = NC * NS

EPT = N_EDGES // NW
K = 125
NCHUNK = 80

H_BINS = 10240
H_R = 80
A_ROWS_T = 632
A_ROWS = A_ROWS_T * NS

_mesh = plsc.VectorSubcoreMesh(core_axis_name="c", subcore_axis_name="s")


def _wid():
  return lax.axis_index("s") * NC + lax.axis_index("c")


@functools.partial(
    pl.kernel,
    out_type=jax.ShapeDtypeStruct((NC, H_R, 128), jnp.float32),
    mesh=_mesh,
    scratch_types=[
        pltpu.VMEM((EPT,), jnp.int32),
        pltpu.VMEM((1, H_R), jnp.int32),
        pltpu.VMEM((H_R, 128), jnp.float32),
        pltpu.VMEM_SHARED((H_R, 128), jnp.float32),
    ],
    compiler_params=pltpu.CompilerParams(needs_layout_passes=False),
)
def _deg_kernel(dst2_hbm, iota_hbm, z_hbm, deg_out, idx_v, iota_v, hist_v,
                hist_sh):
  cid = lax.axis_index("c")
  sid = lax.axis_index("s")
  wid = _wid()
  pltpu.sync_copy(iota_hbm, iota_v)
  pltpu.sync_copy(z_hbm, hist_v)
  pltpu.sync_copy(dst2_hbm.at[wid], idx_v)

  @pl.when(sid == 0)
  def _():
    pltpu.sync_copy(z_hbm, hist_sh)

  ones16 = jnp.ones((16,), jnp.float32)

  def grp(j, carry):
    idx = idx_v[pl.ds(j * 16, 16)]
    row = lax.shift_right_logical(idx, 7)
    col = lax.bitwise_and(idx, 127)
    plsc.addupdate_scatter(hist_v, [row, col], ones16)
    return carry

  lax.fori_loop(0, EPT // 16, grp, 0)
  plsc.subcore_barrier()
  pltpu.sync_copy(hist_v, hist_sh.at[iota_v.at[0]], add=True)
  plsc.subcore_barrier()

  @pl.when(sid == 0)
  def _():
    pltpu.sync_copy(hist_sh, deg_out.at[cid])


def _scale_body(degp_ref, x_ref, v_ref):
  deg = degp_ref[0, :N_NODES] + degp_ref[1, :N_NODES] + 1.0
  dinv = lax.rsqrt(deg)
  v_ref[...] = x_ref[...] * dinv[:, None]


def _scale_call(degp, x):
  return pl.pallas_call(
      _scale_body,
      out_shape=jax.ShapeDtypeStruct((N_NODES, D), jnp.float32),
  )(degp, x)


@functools.partial(
    pl.kernel,
    out_type=jax.ShapeDtypeStruct((NC, A_ROWS, D), jnp.float32),
    mesh=_mesh,
    scratch_types=[
        pltpu.VMEM((NCHUNK // 2, K), jnp.int32),
        pltpu.VMEM((NCHUNK // 2, K), jnp.int32),
        pltpu.VMEM((2, K, D), jnp.float32),
        pltpu.SemaphoreType.DMA,
        pltpu.SemaphoreType.DMA,
        pltpu.VMEM_SHARED((A_ROWS, D), jnp.float32),
    ],
)
def _edge_kernel(src3_hbm, dst3_hbm, v_hbm, z_hbm, acc_out,
                 idxd_v, idxs_v, rows_v, gsem0, gsem1, acc_sh):
  cid = lax.axis_index("c")
  sid = lax.axis_index("s")
  wid = _wid()
  gsem = [gsem0, gsem1]
  HALF = NCHUNK // 2
  pltpu.sync_copy(z_hbm, acc_sh.at[pl.ds(sid * A_ROWS_T, A_ROWS_T)])
  plsc.subcore_barrier()

  for h in range(2):
    pltpu.sync_copy(dst3_hbm.at[wid, pl.ds(h * HALF, HALF)], idxd_v)
    pltpu.sync_copy(src3_hbm.at[wid, pl.ds(h * HALF, HALF)], idxs_v)
    pltpu.async_copy(v_hbm.at[idxs_v.at[0]], rows_v.at[0], gsem0)
    pltpu.async_copy(v_hbm.at[idxs_v.at[1]], rows_v.at[1], gsem1)

    def body(cc, carry):
      for rb in range(2):
        c = cc * 2 + rb
        pltpu.make_async_copy(
            v_hbm.at[idxs_v.at[c]], rows_v.at[rb], gsem[rb]).wait()
        pltpu.sync_copy(rows_v.at[rb], acc_sh.at[idxd_v.at[c]], add=True)

        @pl.when(c + 2 < HALF)
        def _():
          pltpu.async_copy(
              v_hbm.at[idxs_v.at[c + 2]], rows_v.at[rb], gsem[rb])
      return carry

    lax.fori_loop(0, HALF // 2, body, 0)

  plsc.subcore_barrier()
  pltpu.sync_copy(
      acc_sh.at[pl.ds(sid * A_ROWS_T, A_ROWS_T)],
      acc_out.at[cid, pl.ds(sid * A_ROWS_T, A_ROWS_T)],
  )


def _mm_body(x_ref, w_ref, o_ref):
  o_ref[...] = jnp.dot(
      x_ref[...], w_ref[...], preferred_element_type=jnp.float32)


def _mm_call(x, W):
  return pl.pallas_call(
      _mm_body,
      out_shape=jax.ShapeDtypeStruct((N_NODES, D), jnp.float32),
  )(x, W)


def _out_body(accp_ref, degp_ref, xw_ref, b_ref, o_ref):
  deg = degp_ref[0, :N_NODES] + degp_ref[1, :N_NODES] + 1.0
  dinv = lax.rsqrt(deg)
  s = accp_ref[0, :N_NODES] + accp_ref[1, :N_NODES] + xw_ref[...] * dinv[:, None]
  o_ref[...] = s * dinv[:, None] + b_ref[...][None, :]


def _out_call(accp, degp, xw, b):
  return pl.pallas_call(
      _out_body,
      out_shape=jax.ShapeDtypeStruct((N_NODES, D), jnp.float32),
  )(accp, degp, xw, b)


def kernel(x, edge_index, W, b):
  ei = edge_index.astype(jnp.int32)
  src = ei[0]
  dst = ei[1]
  src3 = src.reshape(NW, NCHUNK, K)
  dst3 = dst.reshape(NW, NCHUNK, K)
  dst2 = dst.reshape(NW, EPT)
  iota_r = jnp.arange(H_R, dtype=jnp.int32).reshape(1, H_R)
  z_hist = jnp.zeros((H_R, 128), jnp.float32)
  z_acc = jnp.zeros((A_ROWS_T, D), jnp.float32)

  xw = _mm_call(x, W)
  degp = _deg_kernel(dst2, iota_r, z_hist)
  degf = degp.reshape(NC, H_BINS)
  v = _scale_call(degf, xw)
  accp = _edge_kernel(src3, dst3, v, z_acc)
  return _out_call(accp, degf, xw, b)
